# baseline (speedup 1.0000x reference)
import dataclasses
import functools

import jax
import jax.numpy as jnp
from jax import lax
from jax.experimental import pallas as pl
from jax.experimental.pallas import tpu as pltpu
from jax.experimental.pallas import tpu_sc as plsc

VOCAB = 100000
EMBED_DIM = 128
DEPTH = 17
N_INTERNAL = 2 ** DEPTH - 1
BATCH = 4096
CTX = 8

NC = 2
NS = 16
NW = NC * NS
BPW = BATCH // NW
NCHUNK = BPW * CTX // 128
NG = BPW // 16
NSPLIT = 8
RPS = BPW // NSPLIT


def _sum8(vs):
    a0 = vs[0] + vs[1]
    a1 = vs[2] + vs[3]
    a2 = vs[4] + vs[5]
    a3 = vs[6] + vs[7]
    return (a0 + a1) + (a2 + a3)


def _sum16(vs):
    return _sum8(vs[:8]) + _sum8(vs[8:])


_mesh = plsc.VectorSubcoreMesh(core_axis_name="c", subcore_axis_name="s")

_cp = pltpu.CompilerParams()
if "needs_layout_passes" in pltpu.CompilerParams.__dataclass_fields__:
    _cp = dataclasses.replace(_cp, needs_layout_passes=False)


@functools.partial(
    pl.kernel,
    out_type=[
        jax.ShapeDtypeStruct((NW, DEPTH, BPW), jnp.float32),
        jax.ShapeDtypeStruct((NW, BPW), jnp.int32),
    ],
    mesh=_mesh,
    compiler_params=_cp,
    scratch_types=[
        pltpu.VMEM((NCHUNK, 128), jnp.int32),
        pltpu.VMEM((128, EMBED_DIM), jnp.float32),
        pltpu.VMEM((128, EMBED_DIM), jnp.float32),
        pltpu.VMEM((128, EMBED_DIM), jnp.float32),
        pltpu.VMEM((128, EMBED_DIM), jnp.float32),
        pltpu.VMEM((BPW, EMBED_DIM), jnp.float32),
        pltpu.VMEM((BPW, EMBED_DIM), jnp.float32),
        pltpu.VMEM((NSPLIT, RPS), jnp.int32),
        pltpu.VMEM((16, 17), jnp.float32),
        pltpu.VMEM((DEPTH, BPW), jnp.float32),
        pltpu.VMEM((BPW,), jnp.int32),
        pltpu.SemaphoreType.DMA,
        pltpu.SemaphoreType.DMA,
        pltpu.SemaphoreType.DMA,
        pltpu.SemaphoreType.DMA,
    ],
)
def _hs_kernel(ctx_hbm, emb_hbm, th_hbm, scores_out, leaf_out,
               idx_v, ebuf0, ebuf1, ebuf2, ebuf3, xw_v, th_v, node_v,
               pbuf, scores_v, leaf_v, sem0, sem1, sem2, sem3):
    wid = lax.axis_index("s") * NC + lax.axis_index("c")
    lane = jnp.arange(16, dtype=jnp.int32)

    pltpu.sync_copy(ctx_hbm.at[wid], idx_v)

    ebufs = [ebuf0, ebuf1, ebuf2, ebuf3]
    sems = [sem0, sem1, sem2, sem3]
    handles = [None] * 4
    for c in range(4):
        handles[c] = pltpu.async_copy(emb_hbm.at[pl.ds(c * 128, 128)], ebufs[c], sems[c])
    for c in range(NCHUNK):
        pc = c % 4
        handles[pc].wait()
        buf = ebufs[pc]

        @pl.loop(0, 16)
        def _(b, c=c, buf=buf):
            r0 = b * 8
            bb = c * 16 + b
            for dv in range(8):
                sl = pl.ds(dv * 16, 16)
                s = _sum8([buf[r0 + k, sl] for k in range(8)])
                xw_v[bb, sl] = s

        if c + 4 < NCHUNK:
            handles[pc] = pltpu.async_copy(
                emb_hbm.at[pl.ds((c + 4) * 128, 128)], ebufs[pc], sems[pc])

    for sp in range(NSPLIT):
        for j in range(RPS // 16):
            node_v[sp, pl.ds(j * 16, 16)] = jnp.zeros((16,), jnp.int32)

    @pl.loop(0, DEPTH)
    def _(t):
        cs = [pltpu.async_copy(th_hbm.at[node_v.at[sp]],
                               th_v.at[pl.ds(sp * RPS, RPS)], sems[sp % 4])
              for sp in range(NSPLIT)]
        for c in cs:
            c.wait()

        @pl.loop(0, NG)
        def _(g):
            score = th_v[g, pl.ds(0, 16)] * xw_v[g, pl.ds(0, 16)]
            scores_v[t, pl.ds(g * 16, 16)] = score
            sp = g // (NG // NSPLIT)
            off = (g % (NG // NSPLIT)) * 16
            nd = node_v[sp, pl.ds(off, 16)]
            node_v[sp, pl.ds(off, 16)] = nd * 2 + jnp.where(score < 0.0, 1, 2)

    @pl.loop(0, NG)
    def _(g):
        sp = g // (NG // NSPLIT)
        off = (g % (NG // NSPLIT)) * 16
        leaf_v[pl.ds(g * 16, 16)] = node_v[sp, pl.ds(off, 16)] - N_INTERNAL

    pltpu.sync_copy(scores_v, scores_out.at[wid])
    pltpu.sync_copy(leaf_v, leaf_out.at[wid])


@jax.jit
def kernel(context_vector, embeddings, thetas):
    ctx3 = context_vector.astype(jnp.int32).reshape(NW, NCHUNK, 128)
    scores_t, leaf = _hs_kernel(ctx3, embeddings, thetas)
    scores = scores_t.transpose(0, 2, 1).reshape(BATCH, DEPTH)
    leaf_ix = leaf.reshape(BATCH)
    return leaf_ix, scores


# speedup vs baseline: 2.5446x; 2.5446x over previous
import dataclasses
import functools

import jax
import jax.numpy as jnp
from jax import lax
from jax.experimental import pallas as pl
from jax.experimental.pallas import tpu as pltpu
from jax.experimental.pallas import tpu_sc as plsc

VOCAB = 100000
EMBED_DIM = 128
DEPTH = 17
N_INTERNAL = 2 ** DEPTH - 1
BATCH = 4096
CTX = 8

NC = 2
NS = 16
NW = NC * NS
BPW = BATCH // NW
NCHUNK = 16
NG = BPW // 16
SLAB_STEPS = 9
SLAB_ROWS = 2 ** SLAB_STEPS
NEB = 2


def _sum8(vs):
    a0 = vs[0] + vs[1]
    a1 = vs[2] + vs[3]
    a2 = vs[4] + vs[5]
    a3 = vs[6] + vs[7]
    return (a0 + a1) + (a2 + a3)


def _sum16(vs):
    return _sum8(vs[:8]) + _sum8(vs[8:])


_mesh = plsc.VectorSubcoreMesh(core_axis_name="c", subcore_axis_name="s")

_cp = pltpu.CompilerParams()
if "needs_layout_passes" in pltpu.CompilerParams.__dataclass_fields__:
    _cp = dataclasses.replace(_cp, needs_layout_passes=False)


@functools.partial(
    pl.kernel,
    out_type=[
        jax.ShapeDtypeStruct((NW, DEPTH, BPW), jnp.float32),
        jax.ShapeDtypeStruct((NW, BPW), jnp.int32),
    ],
    mesh=_mesh,
    compiler_params=_cp,
    scratch_types=[
        pltpu.VMEM((NCHUNK, 64), jnp.int32),
        pltpu.VMEM((64, EMBED_DIM), jnp.float32),
        pltpu.VMEM((64, EMBED_DIM), jnp.float32),
        pltpu.VMEM((BPW, EMBED_DIM), jnp.float32),
        pltpu.VMEM((SLAB_ROWS, EMBED_DIM), jnp.float32),
        pltpu.VMEM((BPW, EMBED_DIM), jnp.float32),
        pltpu.VMEM((NG, 16), jnp.int32),
        pltpu.VMEM((16, 17), jnp.float32),
        pltpu.VMEM((DEPTH, BPW), jnp.float32),
        pltpu.VMEM((BPW,), jnp.int32),
        pltpu.SemaphoreType.DMA,
        pltpu.SemaphoreType.DMA,
        pltpu.SemaphoreType.DMA,
        pltpu.SemaphoreType.DMA,
    ],
)
def _hs_kernel(ctx_hbm, emb_hbm, th_hbm, scores_out, leaf_out,
               idx_v, ebuf0, ebuf1, xw_v, th_slab, th_ind, node_v,
               pbuf, scores_v, leaf_v, sem0, sem1, sem2, sem3):
    wid = lax.axis_index("s") * NC + lax.axis_index("c")
    lane = jnp.arange(16, dtype=jnp.int32)

    slab_cp = pltpu.async_copy(th_hbm.at[pl.ds(0, SLAB_ROWS)], th_slab, sem3)

    pltpu.sync_copy(ctx_hbm.at[wid], idx_v)
    ebufs = [ebuf0, ebuf1]
    sems = [sem0, sem1, sem2, sem3]
    handles = [None] * NEB
    for c in range(NEB):
        handles[c] = pltpu.async_copy(emb_hbm.at[idx_v.at[c]], ebufs[c], sems[c])
    for c in range(NCHUNK):
        pc = c % NEB
        handles[pc].wait()
        buf = ebufs[pc]

        @pl.loop(0, 8)
        def _(b, c=c, buf=buf):
            r0 = b * 8
            bb = c * 8 + b
            for dv in range(8):
                sl = pl.ds(dv * 16, 16)
                xw_v[bb, sl] = _sum8([buf[r0 + k, sl] for k in range(8)])

        if c + NEB < NCHUNK:
            handles[pc] = pltpu.async_copy(
                emb_hbm.at[idx_v.at[c + NEB]], ebufs[pc], sems[pc])

    for g in range(NG):
        node_v[g, pl.ds(0, 16)] = jnp.zeros((16,), jnp.int32)

    def group_compute(t, g, row_of_b, th_buf):
        nd_vec = node_v[g, pl.ds(0, 16)]
        for b in range(16):
            bb = g * 16 + b
            row = row_of_b(b, bb, nd_vec)
            prods = []
            for dv in range(8):
                sl = pl.ds(dv * 16, 16)
                prods.append(th_buf[row, sl] * xw_v[bb, sl])
            pbuf[b, pl.ds(0, 16)] = _sum8(prods)
        cols = []
        for l in range(16):
            lvec = jnp.full((16,), l, dtype=jnp.int32)
            cols.append(plsc.load_gather(pbuf, [lane, lvec]))
        score = _sum16(cols)
        scores_v[t, pl.ds(g * 16, 16)] = score
        node_v[g, pl.ds(0, 16)] = nd_vec * 2 + jnp.where(score < 0.0, 1, 2)

    slab_cp.wait()

    @pl.loop(0, SLAB_STEPS)
    def _(t):
        @pl.loop(0, NG)
        def _(g, t=t):
            group_compute(t, g, lambda b, bb, nd_vec: nd_vec[b], th_slab)

    @pl.loop(SLAB_STEPS, DEPTH)
    def _(t):
        cs = [pltpu.async_copy(th_hbm.at[node_v.at[g]],
                               th_ind.at[pl.ds(g * 16, 16)], sems[g % 4])
              for g in range(NG)]
        for c in cs:
            c.wait()

        @pl.loop(0, NG)
        def _(g, t=t):
            group_compute(t, g, lambda b, bb, nd_vec: bb, th_ind)

    @pl.loop(0, NG)
    def _(g):
        leaf_v[pl.ds(g * 16, 16)] = node_v[g, pl.ds(0, 16)] - N_INTERNAL

    pltpu.sync_copy(scores_v, scores_out.at[wid])
    pltpu.sync_copy(leaf_v, leaf_out.at[wid])


@jax.jit
def kernel(context_vector, embeddings, thetas):
    ctx3 = context_vector.astype(jnp.int32).reshape(NW, NCHUNK, 64)
    scores_t, leaf = _hs_kernel(ctx3, embeddings, thetas)
    scores = scores_t.transpose(0, 2, 1).reshape(BATCH, DEPTH)
    leaf_ix = leaf.reshape(BATCH)
    return leaf_ix, scores


# speedup vs baseline: 2.8713x; 1.1284x over previous
import dataclasses
import functools

import jax
import jax.numpy as jnp
from jax import lax
from jax.experimental import pallas as pl
from jax.experimental.pallas import tpu as pltpu
from jax.experimental.pallas import tpu_sc as plsc

VOCAB = 100000
EMBED_DIM = 128
DEPTH = 17
N_INTERNAL = 2 ** DEPTH - 1
BATCH = 4096
CTX = 8

NC = 2
NS = 16
NW = NC * NS
BPW = BATCH // NW
NCHUNK = 16
NG = BPW // 16
SLAB_STEPS = 9
SLAB_ROWS = 2 ** SLAB_STEPS
NEB = 2


def _sum8(vs):
    a0 = vs[0] + vs[1]
    a1 = vs[2] + vs[3]
    a2 = vs[4] + vs[5]
    a3 = vs[6] + vs[7]
    return (a0 + a1) + (a2 + a3)


def _sum16(vs):
    return _sum8(vs[:8]) + _sum8(vs[8:])


_mesh = plsc.VectorSubcoreMesh(core_axis_name="c", subcore_axis_name="s")

_cp = pltpu.CompilerParams()
if "needs_layout_passes" in pltpu.CompilerParams.__dataclass_fields__:
    _cp = dataclasses.replace(_cp, needs_layout_passes=False)


@functools.partial(
    pl.kernel,
    out_type=[
        jax.ShapeDtypeStruct((NW, DEPTH, BPW), jnp.float32),
        jax.ShapeDtypeStruct((NW, BPW), jnp.int32),
    ],
    mesh=_mesh,
    compiler_params=_cp,
    scratch_types=[
        pltpu.VMEM((NCHUNK, 64), jnp.int32),
        pltpu.VMEM((64, EMBED_DIM), jnp.float32),
        pltpu.VMEM((64, EMBED_DIM), jnp.float32),
        pltpu.VMEM((BPW, EMBED_DIM), jnp.float32),
        pltpu.VMEM((SLAB_ROWS, EMBED_DIM), jnp.float32),
        pltpu.VMEM((BPW, EMBED_DIM), jnp.float32),
        pltpu.VMEM((NG, 16), jnp.int32),
        pltpu.VMEM((16, 17), jnp.float32),
        pltpu.VMEM((DEPTH, BPW), jnp.float32),
        pltpu.VMEM((BPW,), jnp.int32),
        pltpu.SemaphoreType.DMA,
        pltpu.SemaphoreType.DMA,
        pltpu.SemaphoreType.DMA,
        pltpu.SemaphoreType.DMA,
    ],
)
def _hs_kernel(ctx_hbm, emb_hbm, th_hbm, scores_out, leaf_out,
               idx_v, ebuf0, ebuf1, xw_v, th_slab, th_ind, node_v,
               pbuf, scores_v, leaf_v, sem0, sem1, sem2, sem3):
    wid = lax.axis_index("s") * NC + lax.axis_index("c")
    lane = jnp.arange(16, dtype=jnp.int32)

    def group_compute(t, g, row_of_b, th_buf):
        nd_vec = node_v[g, pl.ds(0, 16)]
        for b in range(16):
            bb = g * 16 + b
            row = row_of_b(b, bb, nd_vec)
            prods = []
            for dv in range(8):
                sl = pl.ds(dv * 16, 16)
                prods.append(th_buf[row, sl] * xw_v[bb, sl])
            pbuf[b, pl.ds(0, 16)] = _sum8(prods)
        cols = []
        for l in range(16):
            lvec = jnp.full((16,), l, dtype=jnp.int32)
            cols.append(plsc.load_gather(pbuf, [lane, lvec]))
        score = _sum16(cols)
        scores_v[t, pl.ds(g * 16, 16)] = score
        node_v[g, pl.ds(0, 16)] = nd_vec * 2 + jnp.where(score < 0.0, 1, 2)

    pltpu.sync_copy(ctx_hbm.at[wid], idx_v)
    pltpu.async_copy(emb_hbm.at[idx_v.at[0]], ebuf0, sem0)
    pltpu.async_copy(emb_hbm.at[idx_v.at[1]], ebuf1, sem1)
    pltpu.async_copy(th_hbm.at[pl.ds(0, SLAB_ROWS)], th_slab, sem3)

    @pl.loop(0, NCHUNK, step=2)
    def _(c):
        g = c // 2

        def sum_chunk(cc, buf):
            @pl.loop(0, 8)
            def _(b, cc=cc, buf=buf):
                r0 = b * 8
                bb = cc * 8 + b
                for dv in range(8):
                    sl = pl.ds(dv * 16, 16)
                    xw_v[bb, sl] = _sum8([buf[r0 + k, sl] for k in range(8)])

        pltpu.make_async_copy(emb_hbm.at[idx_v.at[c]], ebuf0, sem0).wait()
        sum_chunk(c, ebuf0)

        @pl.when(c + 2 < NCHUNK)
        def _(c=c):
            pltpu.async_copy(emb_hbm.at[idx_v.at[c + 2]], ebuf0, sem0)

        pltpu.make_async_copy(emb_hbm.at[idx_v.at[c + 1]], ebuf1, sem1).wait()
        sum_chunk(c + 1, ebuf1)

        @pl.when(c + 3 < NCHUNK)
        def _(c=c):
            pltpu.async_copy(emb_hbm.at[idx_v.at[c + 3]], ebuf1, sem1)

        @pl.when(c == 0)
        def _():
            pltpu.make_async_copy(
                th_hbm.at[pl.ds(0, SLAB_ROWS)], th_slab, sem3).wait()

        node_v[g, pl.ds(0, 16)] = jnp.zeros((16,), jnp.int32)

        @pl.loop(0, SLAB_STEPS)
        def _(t, g=g):
            group_compute(t, g, lambda b, bb, nd_vec: nd_vec[b], th_slab)

    @pl.loop(SLAB_STEPS, DEPTH)
    def _(t):
        @pl.loop(0, NG)
        def _(g):
            pltpu.async_copy(th_hbm.at[node_v.at[g]],
                             th_ind.at[pl.ds(g * 16, 16)], sem2)

        @pl.loop(0, NG)
        def _(g, t=t):
            pltpu.make_async_copy(th_hbm.at[node_v.at[g]],
                                  th_ind.at[pl.ds(g * 16, 16)], sem2).wait()
            group_compute(t, g, lambda b, bb, nd_vec: bb, th_ind)

    @pl.loop(0, NG)
    def _(g):
        leaf_v[pl.ds(g * 16, 16)] = node_v[g, pl.ds(0, 16)] - N_INTERNAL

    pltpu.sync_copy(scores_v, scores_out.at[wid])
    pltpu.sync_copy(leaf_v, leaf_out.at[wid])


@jax.jit
def kernel(context_vector, embeddings, thetas):
    ctx3 = context_vector.astype(jnp.int32).reshape(NW, NCHUNK, 64)
    scores_t, leaf = _hs_kernel(ctx3, embeddings, thetas)
    scores = scores_t.transpose(0, 2, 1).reshape(BATCH, DEPTH)
    leaf_ix = leaf.reshape(BATCH)
    return leaf_ix, scores


# speedup vs baseline: 2.9079x; 1.0128x over previous
import dataclasses
import functools

import jax
import jax.numpy as jnp
from jax import lax
from jax.experimental import pallas as pl
from jax.experimental.pallas import tpu as pltpu
from jax.experimental.pallas import tpu_sc as plsc

VOCAB = 100000
EMBED_DIM = 128
DEPTH = 17
N_INTERNAL = 2 ** DEPTH - 1
BATCH = 4096
CTX = 8

NC = 2
NS = 16
NW = NC * NS
BPW = BATCH // NW
NCHUNK = 16
NG = BPW // 16
SLAB_STEPS = 9
SLAB_ROWS = 2 ** SLAB_STEPS
NEB = 2


def _sum8(vs):
    a0 = vs[0] + vs[1]
    a1 = vs[2] + vs[3]
    a2 = vs[4] + vs[5]
    a3 = vs[6] + vs[7]
    return (a0 + a1) + (a2 + a3)


def _sum16(vs):
    return _sum8(vs[:8]) + _sum8(vs[8:])


_mesh = plsc.VectorSubcoreMesh(core_axis_name="c", subcore_axis_name="s")

_cp = pltpu.CompilerParams()
if "needs_layout_passes" in pltpu.CompilerParams.__dataclass_fields__:
    _cp = dataclasses.replace(_cp, needs_layout_passes=False)


@functools.partial(
    pl.kernel,
    out_type=[
        jax.ShapeDtypeStruct((NW, DEPTH, BPW), jnp.float32),
        jax.ShapeDtypeStruct((NW, BPW), jnp.int32),
    ],
    mesh=_mesh,
    compiler_params=_cp,
    scratch_types=[
        pltpu.VMEM((NCHUNK, 64), jnp.int32),
        pltpu.VMEM((64, EMBED_DIM), jnp.float32),
        pltpu.VMEM((64, EMBED_DIM), jnp.float32),
        pltpu.VMEM((BPW, EMBED_DIM), jnp.float32),
        pltpu.VMEM((SLAB_ROWS, EMBED_DIM), jnp.float32),
        pltpu.VMEM((64, EMBED_DIM), jnp.float32),
        pltpu.VMEM_SHARED((2 * SLAB_ROWS, EMBED_DIM), jnp.float32),
        pltpu.VMEM((NG, 16), jnp.int32),
        pltpu.VMEM((16, 17), jnp.float32),
        pltpu.VMEM((DEPTH, BPW), jnp.float32),
        pltpu.VMEM((BPW,), jnp.int32),
        pltpu.SemaphoreType.DMA,
        pltpu.SemaphoreType.DMA,
        pltpu.SemaphoreType.DMA,
        pltpu.SemaphoreType.DMA,
    ],
)
def _hs_kernel(ctx_hbm, emb_hbm, th_hbm, scores_out, leaf_out,
               idx_v, ebuf0, ebuf1, xw_v, th_slab, th_ind, th_sp, node_v,
               pbuf, scores_v, leaf_v, sem0, sem1, sem2, sem3):
    sid = lax.axis_index("s")
    wid = sid * NC + lax.axis_index("c")
    lane = jnp.arange(16, dtype=jnp.int32)

    def group_compute(t, g, row_of_b, th_buf):
        nd_vec = node_v[g, pl.ds(0, 16)]
        for b in range(16):
            bb = g * 16 + b
            row = row_of_b(b, bb, nd_vec)
            prods = []
            for dv in range(8):
                sl = pl.ds(dv * 16, 16)
                prods.append(th_buf[row, sl] * xw_v[bb, sl])
            pbuf[b, pl.ds(0, 16)] = _sum8(prods)
        cols = []
        for l in range(16):
            lvec = jnp.full((16,), l, dtype=jnp.int32)
            cols.append(plsc.load_gather(pbuf, [lane, lvec]))
        score = _sum16(cols)
        scores_v[t, pl.ds(g * 16, 16)] = score
        node_v[g, pl.ds(0, 16)] = nd_vec * 2 + jnp.where(score < 0.0, 1, 2)

    pltpu.sync_copy(ctx_hbm.at[wid], idx_v)
    pltpu.async_copy(emb_hbm.at[idx_v.at[0]], ebuf0, sem0)
    pltpu.async_copy(emb_hbm.at[idx_v.at[1]], ebuf1, sem1)
    pltpu.async_copy(th_hbm.at[pl.ds(sid * 64, 64)],
                     th_sp.at[pl.ds(sid * 64, 64)], sem3)

    @pl.loop(0, NCHUNK, step=2)
    def _(c):
        g = c // 2

        def sum_chunk(cc, buf):
            @pl.loop(0, 8)
            def _(b, cc=cc, buf=buf):
                r0 = b * 8
                bb = cc * 8 + b
                for dv in range(8):
                    sl = pl.ds(dv * 16, 16)
                    xw_v[bb, sl] = _sum8([buf[r0 + k, sl] for k in range(8)])

        pltpu.make_async_copy(emb_hbm.at[idx_v.at[c]], ebuf0, sem0).wait()
        sum_chunk(c, ebuf0)

        @pl.when(c + 2 < NCHUNK)
        def _(c=c):
            pltpu.async_copy(emb_hbm.at[idx_v.at[c + 2]], ebuf0, sem0)

        pltpu.make_async_copy(emb_hbm.at[idx_v.at[c + 1]], ebuf1, sem1).wait()
        sum_chunk(c + 1, ebuf1)

        @pl.when(c + 3 < NCHUNK)
        def _(c=c):
            pltpu.async_copy(emb_hbm.at[idx_v.at[c + 3]], ebuf1, sem1)

        @pl.when(c == 0)
        def _():
            pltpu.make_async_copy(th_hbm.at[pl.ds(sid * 64, 64)],
                                  th_sp.at[pl.ds(sid * 64, 64)], sem3).wait()
            plsc.subcore_barrier()
            pltpu.sync_copy(th_sp.at[pl.ds(0, SLAB_ROWS)], th_slab)

        node_v[g, pl.ds(0, 16)] = jnp.zeros((16,), jnp.int32)

        @pl.loop(0, SLAB_STEPS)
        def _(t, g=g):
            group_compute(t, g, lambda b, bb, nd_vec: nd_vec[b], th_slab)

    def ind_step(t, src_ref):
        @pl.loop(0, 4)
        def _(g):
            pltpu.async_copy(src_ref.at[node_v.at[g]],
                             th_ind.at[pl.ds((g % 4) * 16, 16)], sem2)

        @pl.loop(0, NG)
        def _(g, t=t):
            slot = pl.ds((g % 4) * 16, 16)
            pltpu.make_async_copy(src_ref.at[node_v.at[g]],
                                  th_ind.at[slot], sem2).wait()
            group_compute(t, g, lambda b, bb, nd_vec: (bb % 64), th_ind)

            @pl.when(g + 4 < NG)
            def _(g=g):
                pltpu.async_copy(src_ref.at[node_v.at[g + 4]],
                                 th_ind.at[pl.ds((g % 4) * 16, 16)], sem2)

    ind_step(SLAB_STEPS, th_sp)

    @pl.loop(SLAB_STEPS + 1, DEPTH)
    def _(t):
        ind_step(t, th_hbm)

    @pl.loop(0, NG)
    def _(g):
        leaf_v[pl.ds(g * 16, 16)] = node_v[g, pl.ds(0, 16)] - N_INTERNAL

    pltpu.sync_copy(scores_v, scores_out.at[wid])
    pltpu.sync_copy(leaf_v, leaf_out.at[wid])


@jax.jit
def kernel(context_vector, embeddings, thetas):
    ctx3 = context_vector.astype(jnp.int32).reshape(NW, NCHUNK, 64)
    scores_t, leaf = _hs_kernel(ctx3, embeddings, thetas)
    scores = scores_t.transpose(0, 2, 1).reshape(BATCH, DEPTH)
    leaf_ix = leaf.reshape(BATCH)
    return leaf_ix, scores


# speedup vs baseline: 3.0611x; 1.0527x over previous
import dataclasses
import functools

import jax
import jax.numpy as jnp
from jax import lax
from jax.experimental import pallas as pl
from jax.experimental.pallas import tpu as pltpu
from jax.experimental.pallas import tpu_sc as plsc

VOCAB = 100000
EMBED_DIM = 128
DEPTH = 17
N_INTERNAL = 2 ** DEPTH - 1
BATCH = 4096
CTX = 8

NC = 2
NS = 16
NW = NC * NS
BPW = BATCH // NW
NCHUNK = 16
NG = BPW // 16
SLAB_STEPS = 9
SLAB_ROWS = 2 ** SLAB_STEPS
NEB = 2


def _sum8(vs):
    a0 = vs[0] + vs[1]
    a1 = vs[2] + vs[3]
    a2 = vs[4] + vs[5]
    a3 = vs[6] + vs[7]
    return (a0 + a1) + (a2 + a3)


def _sum16(vs):
    return _sum8(vs[:8]) + _sum8(vs[8:])


_mesh = plsc.VectorSubcoreMesh(core_axis_name="c", subcore_axis_name="s")

_cp = pltpu.CompilerParams()
if "needs_layout_passes" in pltpu.CompilerParams.__dataclass_fields__:
    _cp = dataclasses.replace(_cp, needs_layout_passes=False)


@functools.partial(
    pl.kernel,
    out_type=[
        jax.ShapeDtypeStruct((NW, DEPTH, BPW), jnp.float32),
        jax.ShapeDtypeStruct((NW, BPW), jnp.int32),
    ],
    mesh=_mesh,
    compiler_params=_cp,
    scratch_types=[
        pltpu.VMEM((NCHUNK, 64), jnp.int32),
        pltpu.VMEM((64, EMBED_DIM), jnp.float32),
        pltpu.VMEM((64, EMBED_DIM), jnp.float32),
        pltpu.VMEM((BPW, EMBED_DIM), jnp.float32),
        pltpu.VMEM((SLAB_ROWS, EMBED_DIM), jnp.float32),
        pltpu.VMEM((64, EMBED_DIM), jnp.float32),
        pltpu.VMEM_SHARED((2 * SLAB_ROWS, EMBED_DIM), jnp.float32),
        pltpu.VMEM((NG, 16), jnp.int32),
        pltpu.VMEM((16, 17), jnp.float32),
        pltpu.VMEM((DEPTH, BPW), jnp.float32),
        pltpu.VMEM((BPW,), jnp.int32),
        pltpu.SemaphoreType.DMA,
        pltpu.SemaphoreType.DMA,
        pltpu.SemaphoreType.DMA,
        pltpu.SemaphoreType.DMA,
    ],
)
def _hs_kernel(ctx_hbm, emb_hbm, th_hbm, scores_out, leaf_out,
               idx_v, ebuf0, ebuf1, xw_v, th_slab, th_ind, th_sp, node_v,
               pbuf, scores_v, leaf_v, sem0, sem1, sem2, sem3):
    sid = lax.axis_index("s")
    wid = sid * NC + lax.axis_index("c")
    lane = jnp.arange(16, dtype=jnp.int32)

    def group_compute(t, g, row_of_b, th_buf):
        nd_vec = node_v[g, pl.ds(0, 16)]
        for b in range(16):
            bb = g * 16 + b
            row = row_of_b(b, bb, nd_vec)
            prods = []
            for dv in range(8):
                sl = pl.ds(dv * 16, 16)
                prods.append(th_buf[row, sl] * xw_v[bb, sl])
            pbuf[b, pl.ds(0, 16)] = _sum8(prods)
        cols = []
        for l in range(16):
            lvec = jnp.full((16,), l, dtype=jnp.int32)
            cols.append(plsc.load_gather(pbuf, [lane, lvec]))
        score = _sum16(cols)
        scores_v[t, pl.ds(g * 16, 16)] = score
        node_v[g, pl.ds(0, 16)] = nd_vec * 2 + jnp.where(score < 0.0, 1, 2)

    pltpu.sync_copy(ctx_hbm.at[wid], idx_v)
    pltpu.async_copy(emb_hbm.at[idx_v.at[0]], ebuf0, sem0)
    pltpu.async_copy(emb_hbm.at[idx_v.at[1]], ebuf1, sem1)
    pltpu.async_copy(th_hbm.at[pl.ds(sid * 64, 64)],
                     th_sp.at[pl.ds(sid * 64, 64)], sem3)

    @pl.loop(0, NCHUNK, step=2)
    def _(c):
        g = c // 2

        def sum_chunk(cc, buf):
            @pl.loop(0, 8)
            def _(b, cc=cc, buf=buf):
                r0 = b * 8
                bb = cc * 8 + b
                for dv in range(8):
                    sl = pl.ds(dv * 16, 16)
                    xw_v[bb, sl] = _sum8([buf[r0 + k, sl] for k in range(8)])

        pltpu.make_async_copy(emb_hbm.at[idx_v.at[c]], ebuf0, sem0).wait()
        sum_chunk(c, ebuf0)

        @pl.when(c + 2 < NCHUNK)
        def _(c=c):
            pltpu.async_copy(emb_hbm.at[idx_v.at[c + 2]], ebuf0, sem0)

        pltpu.make_async_copy(emb_hbm.at[idx_v.at[c + 1]], ebuf1, sem1).wait()
        sum_chunk(c + 1, ebuf1)

        @pl.when(c + 3 < NCHUNK)
        def _(c=c):
            pltpu.async_copy(emb_hbm.at[idx_v.at[c + 3]], ebuf1, sem1)

        @pl.when(c == 0)
        def _():
            pltpu.make_async_copy(th_hbm.at[pl.ds(sid * 64, 64)],
                                  th_sp.at[pl.ds(sid * 64, 64)], sem3).wait()
            plsc.subcore_barrier()
            pltpu.sync_copy(th_sp.at[pl.ds(0, SLAB_ROWS)], th_slab)

        node_v[g, pl.ds(0, 16)] = jnp.zeros((16,), jnp.int32)

        @pl.loop(0, SLAB_STEPS)
        def _(t, g=g):
            group_compute(t, g, lambda b, bb, nd_vec: nd_vec[b], th_slab)

    def ind_step(t, src_ref, nxt_src, nxt_guard):
        @pl.loop(0, NG)
        def _(g, t=t):
            slot = pl.ds((g % 4) * 16, 16)
            pltpu.make_async_copy(src_ref.at[node_v.at[g]],
                                  th_ind.at[slot], sem2).wait()
            group_compute(t, g, lambda b, bb, nd_vec: (bb % 64), th_ind)

            @pl.when(g + 4 < NG)
            def _(g=g):
                pltpu.async_copy(src_ref.at[node_v.at[g + 4]],
                                 th_ind.at[pl.ds((g % 4) * 16, 16)], sem2)

            @pl.when((g >= 4) & nxt_guard)
            def _(g=g):
                pltpu.async_copy(nxt_src.at[node_v.at[g - 4]],
                                 th_ind.at[pl.ds((g % 4) * 16, 16)], sem2)

    @pl.loop(0, 4)
    def _(g):
        pltpu.async_copy(th_sp.at[node_v.at[g]],
                         th_ind.at[pl.ds((g % 4) * 16, 16)], sem2)

    ind_step(SLAB_STEPS, th_sp, th_hbm, jnp.bool_(True))

    @pl.loop(SLAB_STEPS + 1, DEPTH)
    def _(t):
        ind_step(t, th_hbm, th_hbm, t < DEPTH - 1)

    @pl.loop(0, NG)
    def _(g):
        leaf_v[pl.ds(g * 16, 16)] = node_v[g, pl.ds(0, 16)] - N_INTERNAL

    pltpu.sync_copy(scores_v, scores_out.at[wid])
    pltpu.sync_copy(leaf_v, leaf_out.at[wid])


@jax.jit
def kernel(context_vector, embeddings, thetas):
    ctx3 = context_vector.astype(jnp.int32).reshape(NW, NCHUNK, 64)
    scores_t, leaf = _hs_kernel(ctx3, embeddings, thetas)
    scores = scores_t.transpose(0, 2, 1).reshape(BATCH, DEPTH)
    leaf_ix = leaf.reshape(BATCH)
    return leaf_ix, scores


# speedup vs baseline: 3.0653x; 1.0014x over previous
import dataclasses
import functools

import jax
import jax.numpy as jnp
from jax import lax
from jax.experimental import pallas as pl
from jax.experimental.pallas import tpu as pltpu
from jax.experimental.pallas import tpu_sc as plsc

VOCAB = 100000
EMBED_DIM = 128
DEPTH = 17
N_INTERNAL = 2 ** DEPTH - 1
BATCH = 4096
CTX = 8

NC = 2
NS = 16
NW = NC * NS
BPW = BATCH // NW
NCHUNK = 16
NG = BPW // 16
SLAB_STEPS = 9
SLAB_ROWS = 2 ** SLAB_STEPS
NEB = 2


def _sum8(vs):
    a0 = vs[0] + vs[1]
    a1 = vs[2] + vs[3]
    a2 = vs[4] + vs[5]
    a3 = vs[6] + vs[7]
    return (a0 + a1) + (a2 + a3)


def _sum16(vs):
    return _sum8(vs[:8]) + _sum8(vs[8:])


_mesh = plsc.VectorSubcoreMesh(core_axis_name="c", subcore_axis_name="s")

_cp = pltpu.CompilerParams()
if "needs_layout_passes" in pltpu.CompilerParams.__dataclass_fields__:
    _cp = dataclasses.replace(_cp, needs_layout_passes=False)


@functools.partial(
    pl.kernel,
    out_type=[
        jax.ShapeDtypeStruct((NW, DEPTH, BPW), jnp.float32),
        jax.ShapeDtypeStruct((NW, BPW), jnp.int32),
    ],
    mesh=_mesh,
    compiler_params=_cp,
    scratch_types=[
        pltpu.VMEM((NCHUNK, 64), jnp.int32),
        pltpu.VMEM((64, EMBED_DIM), jnp.float32),
        pltpu.VMEM((64, EMBED_DIM), jnp.float32),
        pltpu.VMEM((BPW, EMBED_DIM), jnp.float32),
        pltpu.VMEM((SLAB_ROWS, EMBED_DIM), jnp.float32),
        pltpu.VMEM((64, EMBED_DIM), jnp.float32),
        pltpu.VMEM_SHARED((2 * SLAB_ROWS, EMBED_DIM), jnp.float32),
        pltpu.VMEM((NG, 16), jnp.int32),
        pltpu.VMEM((16, 17), jnp.float32),
        pltpu.VMEM((DEPTH, BPW), jnp.float32),
        pltpu.VMEM((BPW,), jnp.int32),
        pltpu.SemaphoreType.DMA,
        pltpu.SemaphoreType.DMA,
        pltpu.SemaphoreType.DMA,
        pltpu.SemaphoreType.DMA,
    ],
)
def _hs_kernel(ctx_hbm, emb_hbm, th_hbm, scores_out, leaf_out,
               idx_v, ebuf0, ebuf1, xw_v, th_slab, th_ind, th_sp, node_v,
               pbuf, scores_v, leaf_v, sem0, sem1, sem2, sem3):
    sid = lax.axis_index("s")
    wid = sid * NC + lax.axis_index("c")
    lane = jnp.arange(16, dtype=jnp.int32)

    def group_compute(t, g, row_of_b, th_buf):
        nd_vec = node_v[g, pl.ds(0, 16)]
        for b in range(16):
            bb = g * 16 + b
            row = row_of_b(b, bb, nd_vec)
            prods = []
            for dv in range(8):
                sl = pl.ds(dv * 16, 16)
                prods.append(th_buf[row, sl] * xw_v[bb, sl])
            pbuf[b, pl.ds(0, 16)] = _sum8(prods)
        cols = []
        for l in range(16):
            lvec = jnp.full((16,), l, dtype=jnp.int32)
            cols.append(plsc.load_gather(pbuf, [lane, lvec]))
        score = _sum16(cols)
        scores_v[t, pl.ds(g * 16, 16)] = score
        node_v[g, pl.ds(0, 16)] = nd_vec * 2 + jnp.where(score < 0.0, 1, 2)

    pltpu.sync_copy(ctx_hbm.at[wid], idx_v)
    pltpu.async_copy(emb_hbm.at[idx_v.at[0]], ebuf0, sem0)
    pltpu.async_copy(emb_hbm.at[idx_v.at[1]], ebuf1, sem1)
    pltpu.async_copy(th_hbm.at[pl.ds(sid * 64, 64)],
                     th_sp.at[pl.ds(sid * 64, 64)], sem3)

    @pl.loop(0, NCHUNK, step=2)
    def _(c):
        g = c // 2

        def sum_chunk(cc, buf):
            @pl.loop(0, 8)
            def _(b, cc=cc, buf=buf):
                r0 = b * 8
                bb = cc * 8 + b
                for dv in range(8):
                    sl = pl.ds(dv * 16, 16)
                    xw_v[bb, sl] = _sum8([buf[r0 + k, sl] for k in range(8)])

        pltpu.make_async_copy(emb_hbm.at[idx_v.at[c]], ebuf0, sem0).wait()
        sum_chunk(c, ebuf0)

        @pl.when(c + 2 < NCHUNK)
        def _(c=c):
            pltpu.async_copy(emb_hbm.at[idx_v.at[c + 2]], ebuf0, sem0)

        pltpu.make_async_copy(emb_hbm.at[idx_v.at[c + 1]], ebuf1, sem1).wait()
        sum_chunk(c + 1, ebuf1)

        @pl.when(c + 3 < NCHUNK)
        def _(c=c):
            pltpu.async_copy(emb_hbm.at[idx_v.at[c + 3]], ebuf1, sem1)

        @pl.when(c == 0)
        def _():
            pltpu.make_async_copy(th_hbm.at[pl.ds(sid * 64, 64)],
                                  th_sp.at[pl.ds(sid * 64, 64)], sem3).wait()
            plsc.subcore_barrier()
            pltpu.sync_copy(th_sp.at[pl.ds(0, SLAB_ROWS)], th_slab)

        node_v[g, pl.ds(0, 16)] = jnp.zeros((16,), jnp.int32)

        @pl.loop(0, SLAB_STEPS)
        def _(t, g=g):
            group_compute(t, g, lambda b, bb, nd_vec: nd_vec[b], th_slab)

        @pl.when(g < 4)
        def _(g=g):
            pltpu.async_copy(th_sp.at[node_v.at[g]],
                             th_ind.at[pl.ds((g % 4) * 16, 16)], sem2)

    def ind_step(t, src_ref, nxt_src, nxt_guard):
        @pl.loop(0, NG)
        def _(g, t=t):
            slot = pl.ds((g % 4) * 16, 16)
            pltpu.make_async_copy(src_ref.at[node_v.at[g]],
                                  th_ind.at[slot], sem2).wait()
            group_compute(t, g, lambda b, bb, nd_vec: (bb % 64), th_ind)

            @pl.when(g + 4 < NG)
            def _(g=g):
                pltpu.async_copy(src_ref.at[node_v.at[g + 4]],
                                 th_ind.at[pl.ds((g % 4) * 16, 16)], sem2)

            @pl.when((g >= 4) & nxt_guard)
            def _(g=g):
                pltpu.async_copy(nxt_src.at[node_v.at[g - 4]],
                                 th_ind.at[pl.ds((g % 4) * 16, 16)], sem2)

    ind_step(SLAB_STEPS, th_sp, th_hbm, jnp.bool_(True))

    @pl.loop(SLAB_STEPS + 1, DEPTH)
    def _(t):
        ind_step(t, th_hbm, th_hbm, t < DEPTH - 1)

    @pl.loop(0, NG)
    def _(g):
        leaf_v[pl.ds(g * 16, 16)] = node_v[g, pl.ds(0, 16)] - N_INTERNAL

    pltpu.sync_copy(scores_v, scores_out.at[wid])
    pltpu.sync_copy(leaf_v, leaf_out.at[wid])


@jax.jit
def kernel(context_vector, embeddings, thetas):
    ctx3 = context_vector.astype(jnp.int32).reshape(NW, NCHUNK, 64)
    scores_t, leaf = _hs_kernel(ctx3, embeddings, thetas)
    scores = scores_t.transpose(0, 2, 1).reshape(BATCH, DEPTH)
    leaf_ix = leaf.reshape(BATCH)
    return leaf_ix, scores
